# baseline (speedup 1.0000x reference)
import functools

import jax
import jax.numpy as jnp
from jax import lax
from jax.experimental import pallas as pl
from jax.experimental.pallas import tpu as pltpu, tpu_sc as plsc

NUM_CORES = 2
NUM_SUBCORES = 16
LANES = 16
NUM_WORKERS = NUM_CORES * NUM_SUBCORES

BATCH = 4
SEQ_LEN = 2048
D_MODEL = 1024
N_TOK = BATCH * SEQ_LEN
POS_PER_W = SEQ_LEN // NUM_WORKERS
TOK_PER_W = POS_PER_W * BATCH
CHUNK = 8
N_CHUNKS = TOK_PER_W // CHUNK
CPB = POS_PER_W // CHUNK
NBG = 4
NBO = 2
SCALE = 32.0


@functools.partial(
    pl.kernel,
    out_type=jax.ShapeDtypeStruct((N_TOK, D_MODEL), jnp.float32),
    mesh=plsc.VectorSubcoreMesh(core_axis_name="c", subcore_axis_name="s"),
    scratch_types=[
        pltpu.VMEM((TOK_PER_W,), jnp.int32),
        pltpu.VMEM((CPB, CHUNK, D_MODEL), jnp.float32),
        pltpu.VMEM((NBG, CHUNK, D_MODEL), jnp.float32),
        pltpu.VMEM((NBO, CHUNK, D_MODEL), jnp.float32),
        pltpu.SemaphoreType.DMA((NBG,)),
        pltpu.SemaphoreType.DMA((NBO,)),
        pltpu.SemaphoreType.DMA((CPB,)),
    ],
)
def _emb_sc(xr_hbm, emb1_hbm, emb2_hbm, out_hbm,
            idx_v, p_v, g_v, o_v, sem_g, sem_o, sem_p):
    wid = lax.axis_index("s") * NUM_CORES + lax.axis_index("c")
    pos0 = wid * POS_PER_W

    pltpu.sync_copy(xr_hbm.at[pl.ds(wid * TOK_PER_W, TOK_PER_W)], idx_v)

    def start_gather(c, b):
        pltpu.async_copy(
            emb1_hbm.at[idx_v.at[pl.ds(c * CHUNK, CHUNK)]],
            g_v.at[b], sem_g.at[b])

    def wait_gather(b):
        pltpu.make_async_copy(
            emb1_hbm.at[idx_v.at[pl.ds(0, CHUNK)]],
            g_v.at[b], sem_g.at[b]).wait()

    def wait_out(bo):
        pltpu.make_async_copy(
            o_v.at[bo], out_hbm.at[pl.ds(0, CHUNK)], sem_o.at[bo]).wait()

    for b in range(NBG):
        start_gather(b, b)
    for k in range(CPB):
        pltpu.async_copy(emb2_hbm.at[pl.ds(pos0 + k * CHUNK, CHUNK)],
                         p_v.at[k], sem_p.at[k])

    @pl.loop(0, BATCH)
    def _bt(bt):
        for cc in range(CPB):
            b = cc % NBG
            bo = cc % NBO
            c = bt * CPB + cc
            obase = bt * SEQ_LEN + pos0 + cc * CHUNK

            wait_gather(b)

            @pl.when(bt == 0)
            def _():
                pltpu.make_async_copy(
                    emb2_hbm.at[pl.ds(pos0, CHUNK)],
                    p_v.at[cc], sem_p.at[cc]).wait()

            if cc < NBO:
                @pl.when(bt >= 1)
                def _():
                    wait_out(bo)
            else:
                wait_out(bo)

            @plsc.parallel_loop(0, CHUNK)
            def row_body(i):
                g_row = g_v.at[b].at[i]
                o_row = o_v.at[bo].at[i]
                p_row = p_v.at[cc].at[i]
                for k in range(D_MODEL // LANES):
                    sl = pl.ds(k * LANES, LANES)
                    o_row[sl] = g_row[sl] * SCALE + p_row[sl]

            pltpu.async_copy(
                o_v.at[bo], out_hbm.at[pl.ds(obase, CHUNK)], sem_o.at[bo])

            @pl.when(c + NBG < N_CHUNKS)
            def _():
                start_gather(c + NBG, b)

    for bo in range(NBO):
        wait_out(bo)


def kernel(x, emb1, emb2):
    xr = (x.astype(jnp.int32)
          .reshape(BATCH, NUM_WORKERS, POS_PER_W)
          .transpose(1, 0, 2)
          .reshape(-1))
    out = _emb_sc(xr, emb1, emb2)
    return out.reshape(x.shape[0], x.shape[1], emb1.shape[1])


# speedup vs baseline: 1.0271x; 1.0271x over previous
import functools

import jax
import jax.numpy as jnp
from jax import lax
from jax.experimental import pallas as pl
from jax.experimental.pallas import tpu as pltpu, tpu_sc as plsc

NUM_CORES = 2
NUM_SUBCORES = 16
LANES = 16
NUM_WORKERS = NUM_CORES * NUM_SUBCORES

BATCH = 4
SEQ_LEN = 2048
D_MODEL = 1024
N_TOK = BATCH * SEQ_LEN
POS_PER_W = SEQ_LEN // NUM_WORKERS
TOK_PER_W = POS_PER_W * BATCH
CHUNK = 16
N_CHUNKS = TOK_PER_W // CHUNK
CPB = POS_PER_W // CHUNK
NBG = 3
SCALE = 32.0


@functools.partial(
    pl.kernel,
    out_type=jax.ShapeDtypeStruct((N_TOK, D_MODEL), jnp.float32),
    mesh=plsc.VectorSubcoreMesh(core_axis_name="c", subcore_axis_name="s"),
    scratch_types=[
        pltpu.VMEM((TOK_PER_W,), jnp.int32),
        pltpu.VMEM((CPB, CHUNK, D_MODEL), jnp.float32),
        pltpu.VMEM((NBG, CHUNK, D_MODEL), jnp.float32),
        pltpu.SemaphoreType.DMA((NBG,)),
        pltpu.SemaphoreType.DMA((NBG,)),
        pltpu.SemaphoreType.DMA((CPB,)),
    ],
)
def _emb_sc(xr_hbm, emb1_hbm, emb2_hbm, out_hbm,
            idx_v, p_v, g_v, sem_g, sem_o, sem_p):
    wid = lax.axis_index("s") * NUM_CORES + lax.axis_index("c")
    pos0 = wid * POS_PER_W

    pltpu.sync_copy(xr_hbm.at[pl.ds(wid * TOK_PER_W, TOK_PER_W)], idx_v)

    def start_gather(c):
        b = c % NBG
        pltpu.async_copy(
            emb1_hbm.at[idx_v.at[pl.ds(c * CHUNK, CHUNK)]],
            g_v.at[b], sem_g.at[b])

    def wait_gather(b):
        pltpu.make_async_copy(
            emb1_hbm.at[idx_v.at[pl.ds(0, CHUNK)]],
            g_v.at[b], sem_g.at[b]).wait()

    def wait_out(b):
        pltpu.make_async_copy(
            g_v.at[b], out_hbm.at[pl.ds(0, CHUNK)], sem_o.at[b]).wait()

    start_gather(0)
    start_gather(1)
    for k in range(CPB):
        pltpu.async_copy(emb2_hbm.at[pl.ds(pos0 + k * CHUNK, CHUNK)],
                         p_v.at[k], sem_p.at[k])

    for c in range(N_CHUNKS):
        b = c % NBG
        bt, pb = divmod(c, CPB)
        obase = bt * SEQ_LEN + pos0 + pb * CHUNK

        wait_gather(b)
        if c < CPB:
            pltpu.make_async_copy(
                emb2_hbm.at[pl.ds(pos0, CHUNK)],
                p_v.at[pb], sem_p.at[pb]).wait()

        @plsc.parallel_loop(0, CHUNK)
        def row_body(i):
            g_row = g_v.at[b].at[i]
            p_row = p_v.at[pb].at[i]
            for k in range(D_MODEL // LANES):
                sl = pl.ds(k * LANES, LANES)
                g_row[sl] = g_row[sl] * SCALE + p_row[sl]

        pltpu.async_copy(
            g_v.at[b], out_hbm.at[pl.ds(obase, CHUNK)], sem_o.at[b])

        if c + 2 < N_CHUNKS:
            b2 = (c + 2) % NBG
            if c >= 1:
                wait_out(b2)
            start_gather(c + 2)

    for c in range(N_CHUNKS - NBG, N_CHUNKS):
        wait_out(c % NBG)


def kernel(x, emb1, emb2):
    xr = (x.astype(jnp.int32)
          .reshape(BATCH, NUM_WORKERS, POS_PER_W)
          .transpose(1, 0, 2)
          .reshape(-1))
    out = _emb_sc(xr, emb1, emb2)
    return out.reshape(x.shape[0], x.shape[1], emb1.shape[1])
